# baseline (speedup 1.0000x reference)
import jax
import jax.numpy as jnp
from jax.experimental import pallas as pl
from jax.experimental.pallas import tpu as pltpu

OBS_DIM = 376
ACT_DIM = 17
MIX_NUM = 16
HIDDEN = 256
WIDTH = HIDDEN * MIX_NUM
BATCH = 32
LOG_STD_MAX = 2.0
LOG_STD_MIN = -5.0
EPS = 1e-5

BLK = 512
NB = WIDTH // BLK
NBUF = 4


def _dot_nt(a, b):
    return jax.lax.dot_general(
        a, b, (((1,), (1,)), ((), ())), preferred_element_type=jnp.float32)


def _body(x_ref, Wg_ref, bg_ref, W1_hbm, b1_ref, g1_ref, be1_ref,
          W2_hbm, b2_ref, g2_ref, be2_ref, Wm_ref, bm_ref, Ws_ref, bs_ref,
          mean_ref, ls_ref, w1_s, w2_s, z2_s, h1_s, sem1, sem2):
    pltpu.make_async_copy(W1_hbm, w1_s, sem1).start()
    for b in range(NBUF):
        pltpu.make_async_copy(
            W2_hbm.at[pl.ds(b * BLK, BLK), :], w2_s.at[b], sem2.at[b]).start()

    x = x_ref[...]
    glog = _dot_nt(x, Wg_ref[...]) + bg_ref[...]
    iota = jax.lax.broadcasted_iota(jnp.int32, (BATCH, MIX_NUM), 1)
    m1 = jnp.max(glog, axis=1, keepdims=True)
    i1 = jnp.min(jnp.where(glog == m1, iota, MIX_NUM), axis=1,
                 keepdims=True)
    mask1 = iota == i1
    glog2 = jnp.where(mask1, -jnp.inf, glog)
    m2 = jnp.max(glog2, axis=1, keepdims=True)
    i2 = jnp.min(jnp.where(glog2 == m2, iota, MIX_NUM), axis=1,
                 keepdims=True)
    mask2 = iota == i2
    a2 = jnp.exp(m2 - m1)
    w1g = 1.0 / (1.0 + a2)
    w2g = a2 / (1.0 + a2)
    scores = jnp.where(mask1, w1g, 0.0) + jnp.where(mask2, w2g, 0.0)

    pltpu.make_async_copy(W1_hbm, w1_s, sem1).wait()
    z1 = _dot_nt(x, w1_s[...]) + b1_ref[...]
    mu = jnp.mean(z1, axis=1, keepdims=True)
    var = jnp.mean(jnp.square(z1 - mu), axis=1, keepdims=True)
    h1 = (z1 - mu) * jax.lax.rsqrt(var + EPS) * g1_ref[...] + be1_ref[...]
    h1 = jnp.maximum(h1, 0.0)
    h1_s[...] = h1

    for b in range(NB):
        s = b % NBUF
        pltpu.make_async_copy(
            W2_hbm.at[pl.ds(b * BLK, BLK), :], w2_s.at[s], sem2.at[s]).wait()
        z2_s[:, pl.ds(b * BLK, BLK)] = _dot_nt(h1, w2_s[s])
        nxt = b + NBUF
        if nxt < NB:
            pltpu.make_async_copy(
                W2_hbm.at[pl.ds(nxt * BLK, BLK), :], w2_s.at[s],
                sem2.at[s]).start()

    z2 = z2_s[...] + b2_ref[...]
    mu = jnp.mean(z2, axis=1, keepdims=True)
    var = jnp.mean(jnp.square(z2 - mu), axis=1, keepdims=True)
    h2 = (z2 - mu) * jax.lax.rsqrt(var + EPS) * g2_ref[...] + be2_ref[...]
    h2 = jnp.maximum(h2, 0.0)

    t_r = jax.lax.broadcasted_iota(jnp.int32, (MIX_NUM, WIDTH), 0)
    t_c = jax.lax.broadcasted_iota(jnp.int32, (MIX_NUM, WIDTH), 1)
    T = (t_r == (t_c % MIX_NUM)).astype(jnp.float32)
    st = jax.lax.dot_general(
        scores, T, (((1,), (0,)), ((), ())),
        preferred_element_type=jnp.float32)
    w = h2 * st

    s_r = jax.lax.broadcasted_iota(jnp.int32, (WIDTH, HIDDEN), 0)
    s_c = jax.lax.broadcasted_iota(jnp.int32, (WIDTH, HIDDEN), 1)
    SM = jnp.where(s_r // MIX_NUM == s_c, 1.0 / MIX_NUM, 0.0)
    out_h = jax.lax.dot_general(
        w, SM, (((1,), (0,)), ((), ())),
        preferred_element_type=jnp.float32)

    mean_ref[...] = _dot_nt(out_h, Wm_ref[...]) + bm_ref[...]
    v = _dot_nt(out_h, Ws_ref[...]) + bs_ref[...]
    ls_ref[...] = (LOG_STD_MIN
                   + 0.5 * (LOG_STD_MAX - LOG_STD_MIN)
                   * (jnp.tanh(v) + 1.0))


@jax.jit
def kernel(x, Wg, bg, W1, b1, g1, be1, W2, b2, g2, be2, Wm, bm, Ws, bs):
    row = lambda v: v.reshape(1, -1)
    vmem = pl.BlockSpec(memory_space=pltpu.MemorySpace.VMEM)
    hbm = pl.BlockSpec(memory_space=pl.ANY)
    out = pl.pallas_call(
        _body,
        in_specs=[vmem, vmem, vmem, hbm, vmem, vmem, vmem,
                  hbm, vmem, vmem, vmem, vmem, vmem, vmem, vmem],
        out_specs=[vmem, vmem],
        out_shape=[
            jax.ShapeDtypeStruct((BATCH, ACT_DIM), jnp.float32),
            jax.ShapeDtypeStruct((BATCH, ACT_DIM), jnp.float32),
        ],
        scratch_shapes=[
            pltpu.VMEM((WIDTH, OBS_DIM), jnp.float32),
            pltpu.VMEM((NBUF, BLK, WIDTH), jnp.float32),
            pltpu.VMEM((BATCH, WIDTH), jnp.float32),
            pltpu.VMEM((BATCH, WIDTH), jnp.float32),
            pltpu.SemaphoreType.DMA,
            pltpu.SemaphoreType.DMA((NBUF,)),
        ],
    )(x, Wg, row(bg), W1, row(b1), row(g1), row(be1),
      W2, row(b2), row(g2), row(be2), Wm, row(bm), Ws, row(bs))
    return (out[0], out[1])


# speedup vs baseline: 1.0047x; 1.0047x over previous
import jax
import jax.numpy as jnp
from jax.experimental import pallas as pl
from jax.experimental.pallas import tpu as pltpu

OBS_DIM = 376
ACT_DIM = 17
MIX_NUM = 16
HIDDEN = 256
WIDTH = HIDDEN * MIX_NUM
BATCH = 32
LOG_STD_MAX = 2.0
LOG_STD_MIN = -5.0
EPS = 1e-5

BLK = 1024
NB = WIDTH // BLK
ACT2 = 2 * ACT_DIM


def _dot_nt(a, b):
    return jax.lax.dot_general(
        a, b, (((1,), (1,)), ((), ())), preferred_element_type=jnp.float32)


def _dot_nn(a, b):
    return jax.lax.dot_general(
        a, b, (((1,), (0,)), ((), ())), preferred_element_type=jnp.float32)


def _body(x_ref, Wg_ref, bg_ref, W1_ref, b1_ref, g1_ref, be1_ref,
          W2_ref, b2_ref, g2_ref, be2_ref, Wm_ref, bm_ref, Ws_ref, bs_ref,
          mean_ref, ls_ref, h1_s, z2_s, st_s, ab_s, sum_s, sq_s):
    i = pl.program_id(0)

    @pl.when(i == 0)
    def _prologue():
        x = x_ref[...]
        glog = _dot_nt(x, Wg_ref[...]) + bg_ref[...]
        iota = jax.lax.broadcasted_iota(jnp.int32, (BATCH, MIX_NUM), 1)
        m1 = jnp.max(glog, axis=1, keepdims=True)
        i1 = jnp.min(jnp.where(glog == m1, iota, MIX_NUM), axis=1,
                     keepdims=True)
        mask1 = iota == i1
        glog2 = jnp.where(mask1, -jnp.inf, glog)
        m2 = jnp.max(glog2, axis=1, keepdims=True)
        i2 = jnp.min(jnp.where(glog2 == m2, iota, MIX_NUM), axis=1,
                     keepdims=True)
        mask2 = iota == i2
        a2 = jnp.exp(m2 - m1)
        w1g = 1.0 / (1.0 + a2)
        w2g = a2 / (1.0 + a2)
        scores = jnp.where(mask1, w1g, 0.0) + jnp.where(mask2, w2g, 0.0)

        t_r = jax.lax.broadcasted_iota(jnp.int32, (MIX_NUM, WIDTH), 0)
        t_c = jax.lax.broadcasted_iota(jnp.int32, (MIX_NUM, WIDTH), 1)
        T = (t_r == (t_c % MIX_NUM)).astype(jnp.float32)
        st_s[...] = _dot_nn(scores, T)

        s_r = jax.lax.broadcasted_iota(jnp.int32, (WIDTH, HIDDEN), 0)
        s_c = jax.lax.broadcasted_iota(jnp.int32, (WIDTH, HIDDEN), 1)
        SM = jnp.where(s_r // MIX_NUM == s_c, 1.0 / MIX_NUM, 0.0)
        wmws = jnp.concatenate([Wm_ref[...], Ws_ref[...]], axis=0)
        ab_s[...] = _dot_nt(SM, wmws)

        z1 = _dot_nt(x, W1_ref[...]) + b1_ref[...]
        mu = jnp.mean(z1, axis=1, keepdims=True)
        var = jnp.mean(jnp.square(z1 - mu), axis=1, keepdims=True)
        h1 = (z1 - mu) * jax.lax.rsqrt(var + EPS) * g1_ref[...] + be1_ref[...]
        h1_s[...] = jnp.maximum(h1, 0.0)

    zb = _dot_nt(h1_s[...], W2_ref[...]) + b2_ref[:, pl.ds(i * BLK, BLK)]
    z2_s[:, pl.ds(i * BLK, BLK)] = zb
    bsum = jnp.sum(zb, axis=1, keepdims=True)
    bsq = jnp.sum(jnp.square(zb), axis=1, keepdims=True)

    @pl.when(i == 0)
    def _init_stats():
        sum_s[...] = bsum
        sq_s[...] = bsq

    @pl.when(i > 0)
    def _acc_stats():
        sum_s[...] += bsum
        sq_s[...] += bsq

    @pl.when(i == NB - 1)
    def _epilogue():
        mu = sum_s[...] * (1.0 / WIDTH)
        var = sq_s[...] * (1.0 / WIDTH) - jnp.square(mu)
        h2 = (z2_s[...] - mu) * jax.lax.rsqrt(var + EPS) * g2_ref[...] \
            + be2_ref[...]
        w = jnp.maximum(h2, 0.0) * st_s[...]
        mls = _dot_nn(w, ab_s[...])
        mean_ref[...] = mls[:, :ACT_DIM] + bm_ref[...]
        v = mls[:, ACT_DIM:] + bs_ref[...]
        ls_ref[...] = (LOG_STD_MIN
                       + 0.5 * (LOG_STD_MAX - LOG_STD_MIN)
                       * (jnp.tanh(v) + 1.0))


@jax.jit
def kernel(x, Wg, bg, W1, b1, g1, be1, W2, b2, g2, be2, Wm, bm, Ws, bs):
    row = lambda v: v.reshape(1, -1)
    full = lambda shape: pl.BlockSpec(shape, lambda i: (0, 0))
    out = pl.pallas_call(
        _body,
        grid=(NB,),
        in_specs=[
            full((BATCH, OBS_DIM)),
            full((MIX_NUM, OBS_DIM)),
            full((1, MIX_NUM)),
            full((WIDTH, OBS_DIM)),
            full((1, WIDTH)),
            full((1, WIDTH)),
            full((1, WIDTH)),
            pl.BlockSpec((BLK, WIDTH), lambda i: (i, 0)),
            full((1, WIDTH)),
            full((1, WIDTH)),
            full((1, WIDTH)),
            full((ACT_DIM, HIDDEN)),
            full((1, ACT_DIM)),
            full((ACT_DIM, HIDDEN)),
            full((1, ACT_DIM)),
        ],
        out_specs=[full((BATCH, ACT_DIM)), full((BATCH, ACT_DIM))],
        out_shape=[
            jax.ShapeDtypeStruct((BATCH, ACT_DIM), jnp.float32),
            jax.ShapeDtypeStruct((BATCH, ACT_DIM), jnp.float32),
        ],
        scratch_shapes=[
            pltpu.VMEM((BATCH, WIDTH), jnp.float32),
            pltpu.VMEM((BATCH, WIDTH), jnp.float32),
            pltpu.VMEM((BATCH, WIDTH), jnp.float32),
            pltpu.VMEM((WIDTH, ACT2), jnp.float32),
            pltpu.VMEM((BATCH, 1), jnp.float32),
            pltpu.VMEM((BATCH, 1), jnp.float32),
        ],
    )(x, Wg, row(bg), W1, row(b1), row(g1), row(be1),
      W2, row(b2), row(g2), row(be2), Wm, row(bm), Ws, row(bs))
    return (out[0], out[1])


# speedup vs baseline: 1.0304x; 1.0256x over previous
import functools

import jax
import jax.numpy as jnp
from jax.experimental import pallas as pl
from jax.experimental.pallas import tpu as pltpu

OBS_DIM = 376
ACT_DIM = 17
MIX_NUM = 16
HIDDEN = 256
WIDTH = HIDDEN * MIX_NUM
BATCH = 32
LOG_STD_MAX = 2.0
LOG_STD_MIN = -5.0
EPS = 1e-5

BLK = 1024
NSTREAM = 1
NB = WIDTH // BLK
NG = NB // NSTREAM


def _dot_nt(a, b):
    return jax.lax.dot_general(
        a, b, (((1,), (1,)), ((), ())), preferred_element_type=jnp.float32)


def _body(x_ref, Wg_ref, bg_ref, W1_ref, b1_ref, g1_ref, be1_ref,
          *rest):
    W2_refs = rest[:NSTREAM]
    (b2_ref, g2_ref, be2_ref, Wm_ref, bm_ref, Ws_ref, bs_ref,
     mean_ref, ls_ref, h1_s, z2_s, sc_s) = rest[NSTREAM:]
    i = pl.program_id(0)

    @pl.when(i == 0)
    def _prologue():
        x = x_ref[...]
        glog = _dot_nt(x, Wg_ref[...]) + bg_ref[...]
        iota = jax.lax.broadcasted_iota(jnp.int32, (BATCH, MIX_NUM), 1)
        m1 = jnp.max(glog, axis=1, keepdims=True)
        i1 = jnp.min(jnp.where(glog == m1, iota, MIX_NUM), axis=1,
                     keepdims=True)
        mask1 = iota == i1
        glog2 = jnp.where(mask1, -jnp.inf, glog)
        m2 = jnp.max(glog2, axis=1, keepdims=True)
        i2 = jnp.min(jnp.where(glog2 == m2, iota, MIX_NUM), axis=1,
                     keepdims=True)
        mask2 = iota == i2
        a2 = jnp.exp(m2 - m1)
        w1 = 1.0 / (1.0 + a2)
        w2 = a2 / (1.0 + a2)
        sc_s[...] = jnp.where(mask1, w1, 0.0) + jnp.where(mask2, w2, 0.0)

        z1 = _dot_nt(x, W1_ref[...]) + b1_ref[...]
        mu = jnp.mean(z1, axis=1, keepdims=True)
        var = jnp.mean(jnp.square(z1 - mu), axis=1, keepdims=True)
        h1 = (z1 - mu) * jax.lax.rsqrt(var + EPS) * g1_ref[...] + be1_ref[...]
        h1_s[...] = jnp.maximum(h1, 0.0)

    for s in range(NSTREAM):
        z2_s[:, pl.ds((i * NSTREAM + s) * BLK, BLK)] = _dot_nt(
            h1_s[...], W2_refs[s][...])

    @pl.when(i == NG - 1)
    def _epilogue():
        z2 = z2_s[...] + b2_ref[...]
        mu = jnp.mean(z2, axis=1, keepdims=True)
        var = jnp.mean(jnp.square(z2 - mu), axis=1, keepdims=True)
        h2 = (z2 - mu) * jax.lax.rsqrt(var + EPS) * g2_ref[...] + be2_ref[...]
        h2 = jnp.maximum(h2, 0.0)

        t_r = jax.lax.broadcasted_iota(jnp.int32, (MIX_NUM, WIDTH), 0)
        t_c = jax.lax.broadcasted_iota(jnp.int32, (MIX_NUM, WIDTH), 1)
        T = (t_r == (t_c % MIX_NUM)).astype(jnp.float32)
        st = jax.lax.dot_general(
            sc_s[...], T, (((1,), (0,)), ((), ())),
            preferred_element_type=jnp.float32)
        w = h2 * st

        s_r = jax.lax.broadcasted_iota(jnp.int32, (WIDTH, HIDDEN), 0)
        s_c = jax.lax.broadcasted_iota(jnp.int32, (WIDTH, HIDDEN), 1)
        SM = jnp.where(s_r // MIX_NUM == s_c, 1.0 / MIX_NUM, 0.0)
        out_h = jax.lax.dot_general(
            w, SM, (((1,), (0,)), ((), ())),
            preferred_element_type=jnp.float32)

        mean_ref[...] = _dot_nt(out_h, Wm_ref[...]) + bm_ref[...]
        v = _dot_nt(out_h, Ws_ref[...]) + bs_ref[...]
        ls_ref[...] = (LOG_STD_MIN
                       + 0.5 * (LOG_STD_MAX - LOG_STD_MIN)
                       * (jnp.tanh(v) + 1.0))


@jax.jit
def kernel(x, Wg, bg, W1, b1, g1, be1, W2, b2, g2, be2, Wm, bm, Ws, bs):
    row = lambda v: v.reshape(1, -1)
    full = lambda shape: pl.BlockSpec(shape, lambda i: (0, 0))
    out = pl.pallas_call(
        _body,
        grid=(NG,),
        in_specs=[
            full((BATCH, OBS_DIM)),
            full((MIX_NUM, OBS_DIM)),
            full((1, MIX_NUM)),
            full((WIDTH, OBS_DIM)),
            full((1, WIDTH)),
            full((1, WIDTH)),
            full((1, WIDTH)),
        ] + [
            pl.BlockSpec((BLK, WIDTH),
                         functools.partial(lambda s, i: (i * NSTREAM + s, 0), s))
            for s in range(NSTREAM)
        ] + [
            full((1, WIDTH)),
            full((1, WIDTH)),
            full((1, WIDTH)),
            full((ACT_DIM, HIDDEN)),
            full((1, ACT_DIM)),
            full((ACT_DIM, HIDDEN)),
            full((1, ACT_DIM)),
        ],
        out_specs=[full((BATCH, ACT_DIM)), full((BATCH, ACT_DIM))],
        out_shape=[
            jax.ShapeDtypeStruct((BATCH, ACT_DIM), jnp.float32),
            jax.ShapeDtypeStruct((BATCH, ACT_DIM), jnp.float32),
        ],
        scratch_shapes=[
            pltpu.VMEM((BATCH, WIDTH), jnp.float32),
            pltpu.VMEM((BATCH, WIDTH), jnp.float32),
            pltpu.VMEM((BATCH, MIX_NUM), jnp.float32),
        ],
    )(x, Wg, row(bg), W1, row(b1), row(g1), row(be1),
      *([W2] * NSTREAM), row(b2), row(g2), row(be2), Wm, row(bm), Ws, row(bs))
    return (out[0], out[1])


# speedup vs baseline: 1.4038x; 1.3624x over previous
import functools

import jax
import jax.numpy as jnp
from jax.experimental import pallas as pl
from jax.experimental.pallas import tpu as pltpu

OBS_DIM = 376
ACT_DIM = 17
MIX_NUM = 16
HIDDEN = 256
WIDTH = HIDDEN * MIX_NUM
BATCH = 32
LOG_STD_MAX = 2.0
LOG_STD_MIN = -5.0
EPS = 1e-5

BLK = 1024
NSTREAM = 1
NB = WIDTH // BLK
NG = NB // NSTREAM


def _dot_nt(a, b):
    return jax.lax.dot_general(
        a, b, (((1,), (1,)), ((), ())), preferred_element_type=jnp.float32)


def _dot_nn(a, b):
    return jax.lax.dot_general(
        a, b, (((1,), (0,)), ((), ())), preferred_element_type=jnp.float32)


def _body(x_ref, Wg_ref, bg_ref, W1_ref, b1_ref, g1_ref, be1_ref,
          *rest):
    W2_refs = rest[:NSTREAM]
    (b2_ref, g2_ref, be2_ref, Wm_ref, bm_ref, Ws_ref, bs_ref,
     mean_ref, ls_ref, h1_s, z2_s, sc_s) = rest[NSTREAM:]
    i = pl.program_id(0)

    @pl.when(i == 0)
    def _prologue():
        x = x_ref[...]
        glog = _dot_nt(x, Wg_ref[...]) + bg_ref[...]
        iota = jax.lax.broadcasted_iota(jnp.int32, (BATCH, MIX_NUM), 1)
        m1 = jnp.max(glog, axis=1, keepdims=True)
        i1 = jnp.min(jnp.where(glog == m1, iota, MIX_NUM), axis=1,
                     keepdims=True)
        mask1 = iota == i1
        glog2 = jnp.where(mask1, -jnp.inf, glog)
        m2 = jnp.max(glog2, axis=1, keepdims=True)
        i2 = jnp.min(jnp.where(glog2 == m2, iota, MIX_NUM), axis=1,
                     keepdims=True)
        mask2 = iota == i2
        a2 = jnp.exp(m2 - m1)
        w1 = 1.0 / (1.0 + a2)
        w2 = a2 / (1.0 + a2)
        sc_s[...] = jnp.where(mask1, w1, 0.0) + jnp.where(mask2, w2, 0.0)

        z1 = _dot_nn(x, W1_ref[...]) + b1_ref[...]
        mu = jnp.mean(z1, axis=1, keepdims=True)
        var = jnp.mean(jnp.square(z1 - mu), axis=1, keepdims=True)
        h1 = (z1 - mu) * jax.lax.rsqrt(var + EPS) * g1_ref[...] + be1_ref[...]
        h1_s[...] = jnp.maximum(h1, 0.0)

    for s in range(NSTREAM):
        z2_s[:, pl.ds((i * NSTREAM + s) * BLK, BLK)] = _dot_nt(
            h1_s[...], W2_refs[s][...])

    @pl.when(i == NG - 1)
    def _epilogue():
        z2 = z2_s[...] + b2_ref[...]
        mu = jnp.mean(z2, axis=1, keepdims=True)
        var = jnp.mean(jnp.square(z2 - mu), axis=1, keepdims=True)
        h2 = (z2 - mu) * jax.lax.rsqrt(var + EPS) * g2_ref[...] + be2_ref[...]
        h2 = jnp.maximum(h2, 0.0)

        t_r = jax.lax.broadcasted_iota(jnp.int32, (MIX_NUM, WIDTH), 0)
        t_c = jax.lax.broadcasted_iota(jnp.int32, (MIX_NUM, WIDTH), 1)
        T = (t_r == (t_c % MIX_NUM)).astype(jnp.float32)
        st = jax.lax.dot_general(
            sc_s[...], T, (((1,), (0,)), ((), ())),
            preferred_element_type=jnp.float32)
        w = h2 * st

        s_r = jax.lax.broadcasted_iota(jnp.int32, (WIDTH, HIDDEN), 0)
        s_c = jax.lax.broadcasted_iota(jnp.int32, (WIDTH, HIDDEN), 1)
        SM = jnp.where(s_r // MIX_NUM == s_c, 1.0 / MIX_NUM, 0.0)
        out_h = jax.lax.dot_general(
            w, SM, (((1,), (0,)), ((), ())),
            preferred_element_type=jnp.float32)

        mean_ref[...] = _dot_nt(Wm_ref[...], out_h) + bm_ref[...].T
        v = _dot_nt(Ws_ref[...], out_h) + bs_ref[...].T
        ls_ref[...] = (LOG_STD_MIN
                       + 0.5 * (LOG_STD_MAX - LOG_STD_MIN)
                       * (jnp.tanh(v) + 1.0))


@jax.jit
def kernel(x, Wg, bg, W1, b1, g1, be1, W2, b2, g2, be2, Wm, bm, Ws, bs):
    row = lambda v: v.reshape(1, -1)
    full = lambda shape: pl.BlockSpec(shape, lambda i: (0, 0))
    out = pl.pallas_call(
        _body,
        grid=(NG,),
        in_specs=[
            full((BATCH, OBS_DIM)),
            full((MIX_NUM, OBS_DIM)),
            full((1, MIX_NUM)),
            full((OBS_DIM, WIDTH)),
            full((1, WIDTH)),
            full((1, WIDTH)),
            full((1, WIDTH)),
        ] + [
            pl.BlockSpec((BLK, WIDTH),
                         functools.partial(lambda s, i: (i * NSTREAM + s, 0), s))
            for s in range(NSTREAM)
        ] + [
            full((1, WIDTH)),
            full((1, WIDTH)),
            full((1, WIDTH)),
            full((ACT_DIM, HIDDEN)),
            full((1, ACT_DIM)),
            full((ACT_DIM, HIDDEN)),
            full((1, ACT_DIM)),
        ],
        out_specs=[full((ACT_DIM, BATCH)), full((ACT_DIM, BATCH))],
        out_shape=[
            jax.ShapeDtypeStruct((ACT_DIM, BATCH), jnp.float32),
            jax.ShapeDtypeStruct((ACT_DIM, BATCH), jnp.float32),
        ],
        scratch_shapes=[
            pltpu.VMEM((BATCH, WIDTH), jnp.float32),
            pltpu.VMEM((BATCH, WIDTH), jnp.float32),
            pltpu.VMEM((BATCH, MIX_NUM), jnp.float32),
        ],
    )(x, Wg, row(bg), W1.T, row(b1), row(g1), row(be1),
      *([W2] * NSTREAM), row(b2), row(g2), row(be2),
      Wm, row(bm), Ws, row(bs))
    return (out[0].T, out[1].T)
